# baseline (speedup 1.0000x reference)
import jax
import jax.numpy as jnp
from jax import lax
from jax.experimental import pallas as pl
from jax.experimental.pallas import tpu as pltpu
from jax.experimental.pallas import tpu_sc as plsc

N_IDS = 16384
H_DIM = 64


def _make_gather(num_cores, num_workers, b_per_w):
    def body(idx_hbm, table_hbm, out_hbm, idx_v, rows_v, sem):
        wid = lax.axis_index("s") * num_cores + lax.axis_index("c")
        base = wid * b_per_w
        pltpu.sync_copy(idx_hbm.at[pl.ds(base, b_per_w)], idx_v)
        pltpu.async_copy(table_hbm.at[idx_v], rows_v, sem).wait()
        pltpu.sync_copy(rows_v, out_hbm.at[pl.ds(base, b_per_w)])

    return body


def kernel(node_id, table):
    node_id = jnp.reshape(node_id, (N_IDS,)).astype(jnp.int32)
    info = plsc.get_sparse_core_info()
    nc, ns = info.num_cores, info.num_subcores
    nw = nc * ns
    b_per_w = N_IDS // nw
    mesh = plsc.VectorSubcoreMesh(core_axis_name="c", subcore_axis_name="s")
    f = pl.kernel(
        _make_gather(nc, nw, b_per_w),
        mesh=mesh,
        out_type=jax.ShapeDtypeStruct((N_IDS, H_DIM), jnp.float32),
        scratch_types=[
            pltpu.VMEM((b_per_w,), jnp.int32),
            pltpu.VMEM((b_per_w, H_DIM), jnp.float32),
            pltpu.SemaphoreType.DMA,
        ],
        compiler_params=pltpu.CompilerParams(use_tc_tiling_on_sc=False),
    )
    return f(node_id, table)


# speedup vs baseline: 2.5759x; 2.5759x over previous
import jax
import jax.numpy as jnp
from jax import lax
from jax.experimental import pallas as pl
from jax.experimental.pallas import tpu as pltpu
from jax.experimental.pallas import tpu_sc as plsc

N_IDS = 16384
H_DIM = 64
SUB = 8


def _make_body(nc, b_per_w):
    n_groups = b_per_w // 16

    def body(idx_hbm, table_hbm, out_hbm, idx_v, out_v, sem):
        wid = lax.axis_index("s") * nc + lax.axis_index("c")
        base = wid * b_per_w
        pltpu.sync_copy(idx_hbm.at[pl.ds(base, b_per_w)], idx_v)

        def issue_group(g, carry):
            vec = idx_v[pl.ds(g * 16, 16)]
            blk = lax.shift_right_logical(vec, 3)
            sub = vec & 7
            for i in range(16):
                pltpu.async_copy(
                    table_hbm.at[blk[i], sub[i]],
                    out_v.at[g * 16 + i],
                    sem,
                )
            return carry

        lax.fori_loop(0, n_groups, issue_group, 0)
        pltpu.make_async_copy(out_hbm.at[pl.ds(base, b_per_w)], out_v, sem).wait()
        pltpu.sync_copy(out_v, out_hbm.at[pl.ds(base, b_per_w)])

    return body


def kernel(node_id, table):
    node_id = jnp.reshape(node_id, (N_IDS,)).astype(jnp.int32)
    table3 = jnp.reshape(table, (table.shape[0] // SUB, SUB, H_DIM))
    info = plsc.get_sparse_core_info()
    nc, ns = info.num_cores, info.num_subcores
    b_per_w = N_IDS // (nc * ns)
    mesh = plsc.VectorSubcoreMesh(core_axis_name="c", subcore_axis_name="s")
    f = pl.kernel(
        _make_body(nc, b_per_w),
        mesh=mesh,
        out_type=jax.ShapeDtypeStruct((N_IDS, H_DIM), jnp.float32),
        scratch_types=[
            pltpu.VMEM((b_per_w,), jnp.int32),
            pltpu.VMEM((b_per_w, H_DIM), jnp.float32),
            pltpu.SemaphoreType.DMA,
        ],
        compiler_params=pltpu.CompilerParams(needs_layout_passes=False),
    )
    return f(node_id, table3)
